# baseline (speedup 1.0000x reference)
import functools

import jax
import jax.numpy as jnp
from jax import lax
from jax.experimental import pallas as pl
from jax.experimental.pallas import tpu as pltpu
from jax.experimental.pallas import tpu_sc as plsc

NUM_CORES = 2
NUM_SUBCORES = 16
NW = NUM_CORES * NUM_SUBCORES
NBUF = 5
LANES = 16


@jax.jit
def _embed(id2, W):
    B, S = id2.shape
    D = W.shape[1]
    rows_per_w = B // NW
    n_outer = S // NBUF
    assert B % NW == 0 and S % NBUF == 0 and n_outer >= 2
    mesh = plsc.VectorSubcoreMesh(
        core_axis_name="c", subcore_axis_name="s",
        num_cores=NUM_CORES, num_subcores=NUM_SUBCORES)

    @functools.partial(
        pl.kernel,
        mesh=mesh,
        out_type=jax.ShapeDtypeStruct((S * D, B), jnp.float32),
        scratch_types=[
            pltpu.VMEM((rows_per_w, S), jnp.int32),
            pltpu.VMEM((S, rows_per_w), jnp.int32),
        ]
        + [pltpu.VMEM((rows_per_w, D), jnp.float32)] * NBUF
        + [pltpu.VMEM((D, rows_per_w), jnp.float32)] * NBUF
        + [pltpu.SemaphoreType.DMA] * (2 * NBUF),
        compiler_params=pltpu.CompilerParams(
            use_tc_tiling_on_sc=False, needs_layout_passes=False),
    )
    def k(table_hbm, id_hbm, out_hbm, idb, idt, *rest):
        gbuf = rest[:NBUF]
        tbuf = rest[NBUF:2 * NBUF]
        gsem = rest[2 * NBUF:3 * NBUF]
        ssem = rest[3 * NBUF:]
        wid = lax.axis_index("s") * NUM_CORES + lax.axis_index("c")
        r0 = wid * rows_per_w

        pltpu.sync_copy(id_hbm.at[pl.ds(r0, rows_per_w)], idb)

        lane = lax.iota(jnp.int32, LANES)
        row_idx = [lane + g * LANES for g in range(rows_per_w // LANES)]

        def repack(s, carry):
            scol = jnp.full((LANES,), s, jnp.int32)
            for g in range(rows_per_w // LANES):
                v = plsc.load_gather(idb, [row_idx[g], scol])
                idt[s, pl.ds(g * LANES, LANES)] = v
            return carry

        lax.fori_loop(0, S, repack, 0)

        def gather(s, b):
            pltpu.async_copy(table_hbm.at[idt.at[s]], gbuf[b], gsem[b])

        def wait_gather(s, b):
            pltpu.make_async_copy(
                table_hbm.at[idt.at[s]], gbuf[b], gsem[b]).wait()

        def store(s, b):
            pltpu.async_copy(
                tbuf[b],
                out_hbm.at[pl.ds(s * D, D), pl.ds(r0, rows_per_w)],
                ssem[b])

        def wait_store(s, b):
            pltpu.make_async_copy(
                tbuf[b],
                out_hbm.at[pl.ds(s * D, D), pl.ds(r0, rows_per_w)],
                ssem[b]).wait()

        def transpose(b):
            def body(d, carry):
                dcol = jnp.full((LANES,), d, jnp.int32)
                for g in range(rows_per_w // LANES):
                    v = plsc.load_gather(gbuf[b], [row_idx[g], dcol])
                    tbuf[b][d, pl.ds(g * LANES, LANES)] = v
                return carry
            lax.fori_loop(0, D, body, 0)

        for b in range(NBUF):
            gather(b, b)

        def body(g, carry):
            for b in range(NBUF):
                s = g * NBUF + b
                wait_gather(s, b)

                @pl.when(g > 0)
                def _():
                    wait_store(s - NBUF, b)

                transpose(b)
                gather(s + NBUF, b)
                store(s, b)
            return carry

        lax.fori_loop(0, n_outer - 1, body, 0)

        for b in range(NBUF):
            s = (n_outer - 1) * NBUF + b
            wait_gather(s, b)
            wait_store(s - NBUF, b)
            transpose(b)
            store(s, b)
        for b in range(NBUF):
            s = (n_outer - 1) * NBUF + b
            wait_store(s, b)

    out2 = k(W, id2)
    return jnp.transpose(out2.reshape(S, D, B), (2, 0, 1))


def kernel(id, W):
    return _embed(id.astype(jnp.int32), W)


# speedup vs baseline: 2.5939x; 2.5939x over previous
import functools

import jax
import jax.numpy as jnp
from jax import lax
from jax.experimental import pallas as pl
from jax.experimental.pallas import tpu as pltpu
from jax.experimental.pallas import tpu_sc as plsc

NUM_CORES = 2
NUM_SUBCORES = 16
NW = NUM_CORES * NUM_SUBCORES
NBUF = 5
LANES = 16


@jax.jit
def _embed(id2, W):
    B, S = id2.shape
    D = W.shape[1]
    rows_per_w = B // NW
    n_outer = S // NBUF
    assert B % NW == 0 and S % NBUF == 0 and n_outer >= 2
    mesh = plsc.VectorSubcoreMesh(
        core_axis_name="c", subcore_axis_name="s",
        num_cores=NUM_CORES, num_subcores=NUM_SUBCORES)

    @functools.partial(
        pl.kernel,
        mesh=mesh,
        out_type=jax.ShapeDtypeStruct((S * D, B), jnp.float32),
        scratch_types=[
            pltpu.VMEM((rows_per_w, S), jnp.int32),
            pltpu.VMEM((S, rows_per_w), jnp.int32),
        ]
        + [pltpu.VMEM((rows_per_w, D), jnp.float32)] * NBUF
        + [pltpu.VMEM((D, rows_per_w), jnp.float32)] * NBUF
        + [pltpu.SemaphoreType.DMA] * (2 * NBUF),
        compiler_params=pltpu.CompilerParams(
            use_tc_tiling_on_sc=False, needs_layout_passes=False),
    )
    def k(table_hbm, id_hbm, out_hbm, idb, idt, *rest):
        gbuf = rest[:NBUF]
        tbuf = rest[NBUF:2 * NBUF]
        gsem = rest[2 * NBUF:3 * NBUF]
        ssem = rest[3 * NBUF:]
        wid = lax.axis_index("s") * NUM_CORES + lax.axis_index("c")
        r0 = wid * rows_per_w

        pltpu.sync_copy(id_hbm.at[pl.ds(r0, rows_per_w)], idb)

        lane = lax.iota(jnp.int32, LANES)
        row_idx = [lane + g * LANES for g in range(rows_per_w // LANES)]

        def repack(s, carry):
            scol = jnp.full((LANES,), s, jnp.int32)
            for g in range(rows_per_w // LANES):
                v = plsc.load_gather(idb, [row_idx[g], scol])
                idt[s, pl.ds(g * LANES, LANES)] = v
            return carry

        lax.fori_loop(0, S, repack, 0)

        def gather(s, b):
            pltpu.async_copy(table_hbm.at[idt.at[s]], gbuf[b], gsem[b])

        def wait_gather(s, b):
            pltpu.make_async_copy(
                table_hbm.at[idt.at[s]], gbuf[b], gsem[b]).wait()

        def store(s, b):
            pltpu.async_copy(
                tbuf[b],
                out_hbm.at[pl.ds(s * D, D), pl.ds(r0, rows_per_w)],
                ssem[b])

        def wait_store(s, b):
            pltpu.make_async_copy(
                tbuf[b],
                out_hbm.at[pl.ds(s * D, D), pl.ds(r0, rows_per_w)],
                ssem[b]).wait()

        def transpose(b):
            @functools.partial(plsc.parallel_loop, 0, D, unroll=8)
            def _(d):
                dcol = jnp.full((LANES,), d, jnp.int32)
                for g in range(rows_per_w // LANES):
                    v = plsc.load_gather(gbuf[b], [row_idx[g], dcol])
                    tbuf[b][d, pl.ds(g * LANES, LANES)] = v

        for b in range(NBUF):
            gather(b, b)

        def body(g, carry):
            for b in range(NBUF):
                s = g * NBUF + b
                wait_gather(s, b)

                @pl.when(g > 0)
                def _():
                    wait_store(s - NBUF, b)

                transpose(b)
                gather(s + NBUF, b)
                store(s, b)
            return carry

        lax.fori_loop(0, n_outer - 1, body, 0)

        for b in range(NBUF):
            s = (n_outer - 1) * NBUF + b
            wait_gather(s, b)
            wait_store(s - NBUF, b)
            transpose(b)
            store(s, b)
        for b in range(NBUF):
            s = (n_outer - 1) * NBUF + b
            wait_store(s, b)

    out2 = k(W, id2)
    return jnp.transpose(out2.reshape(S, D, B), (2, 0, 1))


def kernel(id, W):
    return _embed(id.astype(jnp.int32), W)
